# Initial kernel scaffold; baseline (speedup 1.0000x reference)
import jax
import jax.numpy as jnp
from jax.experimental import pallas as pl


def kernel(x, x_mask, edge_index, edge_attr, pos, batch, enc_W1, enc_b1, enc_W2, enc_b2, eenc_W, eenc_b, msg_W, msg_b, upd_W, upd_b, eupd_W, eupd_b, dec_W1, dec_b1, dec_W2, dec_b2):
    raise NotImplementedError("write your pallas kernel here")


# speedup vs baseline: 1.5342x; 1.5342x over previous
import functools

import jax
import jax.numpy as jnp
from jax import lax
from jax.experimental import pallas as pl
from jax.experimental.pallas import tpu as pltpu
from jax.experimental.pallas import tpu_sc as plsc

N = 10000
E = 320000
B = 8
D = 128
EC = 32
REPEATS = 4

NCORES = 2
NSUB = 16
NW = NCORES * NSUB
CH = 80
PER_W = E // NW
NCH = PER_W // CH
NPAD = 10240
ROWS_PER_TILE = NPAD // NSUB

_f32 = jnp.float32


_RN = 2000
_GN = N // _RN
_REB = 8000
_GE = E // _REB


def _onehot(b_block):
    return (b_block == lax.broadcasted_iota(jnp.int32, (1, B), 1)).astype(_f32)


def _full(shape):
    return pl.BlockSpec(shape, lambda i: (0,) * len(shape))


def _rows(shape):
    return pl.BlockSpec(shape, lambda i: (i,) + (0,) * (len(shape) - 1))


def _init_node_body(xin_ref, xm_ref, b_ref, W1, b1, W2, b2, Ws, Wd, Ug, Ub, ub,
                    h_ref, hs_ref, hd_ref, gb_ref, cbc_ref, cnt_ref,
                    xg_s, xbc_s, cnt_s, bcc_s):
    i = pl.program_id(0)
    t = jnp.maximum(xin_ref[...] @ W1[...] + b1[...], 0.0)
    h = t @ W2[...] + b2[...]
    h_ref[...] = h
    hs_ref[...] = h @ Ws[...]
    hd_ref[...] = h @ Wd[...]
    oh = _onehot(b_ref[...])
    bc = (xm_ref[...][:, B - 1:B] > 0.5).astype(_f32)
    ones = jnp.ones((_RN, D), _f32)
    ct = lax.dot_general(oh, ones, (((0,), (0,)), ((), ())))
    xg = lax.dot_general(oh, h, (((0,), (0,)), ((), ())))
    ohbc = oh * bc
    bct = lax.dot_general(ohbc, ones, (((0,), (0,)), ((), ())))
    xbc = lax.dot_general(ohbc, h, (((0,), (0,)), ((), ())))

    @pl.when(i == 0)
    def _():
        xg_s[...] = jnp.zeros_like(xg_s)
        xbc_s[...] = jnp.zeros_like(xbc_s)
        cnt_s[...] = jnp.zeros_like(cnt_s)
        bcc_s[...] = jnp.zeros_like(bcc_s)

    xg_s[...] += xg
    xbc_s[...] += xbc
    cnt_s[...] += ct
    bcc_s[...] += bct

    x_graph = xg_s[...] / jnp.maximum(cnt_s[...], 1.0)
    x_BC = xbc_s[...] / jnp.maximum(bcc_s[...], 1.0)
    cBC = x_BC @ Ub[...] + ub[...]
    gb_ref[...] = x_graph @ Ug[...] + cBC
    cbc_ref[...] = cBC
    cnt_ref[...] = cnt_s[...]


def _tc_init_node(xin, x_mask, batch2, W1, b1, W2, b2, Ws, Wd, Ug, Ub, ub):
    out_shapes = (
        jax.ShapeDtypeStruct((N, D), _f32),
        jax.ShapeDtypeStruct((N, D), _f32),
        jax.ShapeDtypeStruct((N, D), _f32),
        jax.ShapeDtypeStruct((B, D), _f32),
        jax.ShapeDtypeStruct((B, D), _f32),
        jax.ShapeDtypeStruct((B, D), _f32),
    )
    return pl.pallas_call(
        _init_node_body,
        grid=(_GN,),
        in_specs=[
            _rows((_RN, D)), _rows((_RN, B)), _rows((_RN, 1)),
            _full((D, D)), _full((1, D)), _full((D, D)), _full((1, D)),
            _full((D, D)), _full((D, D)), _full((D, D)), _full((D, D)),
            _full((1, D)),
        ],
        out_specs=[
            _rows((_RN, D)), _rows((_RN, D)), _rows((_RN, D)),
            _full((B, D)), _full((B, D)), _full((B, D)),
        ],
        out_shape=out_shapes,
        scratch_shapes=[pltpu.VMEM((B, D), _f32)] * 4,
    )(xin, x_mask, batch2, W1, b1, W2, b2, Ws, Wd, Ug, Ub, ub)


def _stage_a_body(h_ref, agg_ref, gb_ref, cnt_ref, cbc_ref, b_ref,
                  Uh, Ua, Us, Ud, Ws, Wd, Ug,
                  hn_ref, hs_ref, hd_ref, us_ref, ud_ref, gbn_ref, xg_s):
    i = pl.program_id(0)
    h = h_ref[...]
    agg = agg_ref[0] + agg_ref[1]
    oh = _onehot(b_ref[...])
    gbx = oh @ gb_ref[...]
    u = jnp.maximum(h @ Uh[...] + agg @ Ua[...] + gbx, 0.0)
    hn = h + u
    hn_ref[...] = hn
    hs_ref[...] = hn @ Ws[...]
    hd_ref[...] = hn @ Wd[...]
    us_ref[...] = hn @ Us[...]
    ud_ref[...] = hn @ Ud[...]
    xg = lax.dot_general(oh, hn, (((0,), (0,)), ((), ())))

    @pl.when(i == 0)
    def _():
        xg_s[...] = jnp.zeros_like(xg_s)

    xg_s[...] += xg
    x_graph = xg_s[...] / jnp.maximum(cnt_ref[...], 1.0)
    gbn_ref[...] = x_graph @ Ug[...] + cbc_ref[...]


def _tc_stage_a(h, agg2, gb, cnt, cbc, batch2, Uh, Ua, Us, Ud, Ws, Wd, Ug):
    out_shapes = (
        jax.ShapeDtypeStruct((N, D), _f32),
        jax.ShapeDtypeStruct((N, D), _f32),
        jax.ShapeDtypeStruct((N, D), _f32),
        jax.ShapeDtypeStruct((N, EC), _f32),
        jax.ShapeDtypeStruct((N, EC), _f32),
        jax.ShapeDtypeStruct((B, D), _f32),
    )
    return pl.pallas_call(
        _stage_a_body,
        grid=(_GN,),
        in_specs=[
            _rows((_RN, D)),
            pl.BlockSpec((2, _RN, D), lambda i: (0, i, 0)),
            _full((B, D)), _full((B, D)), _full((B, D)),
            _rows((_RN, 1)),
            _full((D, D)), _full((D, D)), _full((D, EC)), _full((D, EC)),
            _full((D, D)), _full((D, D)), _full((D, D)),
        ],
        out_specs=[
            _rows((_RN, D)), _rows((_RN, D)), _rows((_RN, D)),
            _rows((_RN, EC)), _rows((_RN, EC)), _full((B, D)),
        ],
        out_shape=out_shapes,
        scratch_shapes=[pltpu.VMEM((B, D), _f32)],
    )(h, agg2, gb, cnt, cbc, batch2, Uh, Ua, Us, Ud, Ws, Wd, Ug)


def _init_edge_body(ea_ref, eW, eb, We, mb, Ue, ub, ee_ref, eu_ref):
    e = jnp.maximum(ea_ref[...] @ eW[...] + eb[...], 0.0)
    ee_ref[...] = e @ We[...] + mb[...]
    eu_ref[...] = e @ Ue[...] + ub[...]


def _tc_init_edge(edge_attr, eW, eb, We, mb, Ue, ub):
    out_shapes = (
        jax.ShapeDtypeStruct((E, D), _f32),
        jax.ShapeDtypeStruct((E, EC), _f32),
    )
    return pl.pallas_call(
        _init_edge_body,
        grid=(_GE,),
        in_specs=[
            _rows((_REB, 4)),
            _full((4, EC)), _full((1, EC)), _full((EC, D)), _full((1, D)),
            _full((EC, EC)), _full((1, EC)),
        ],
        out_specs=[_rows((_REB, D)), _rows((_REB, EC))],
        out_shape=out_shapes,
    )(edge_attr, eW, eb, We, mb, Ue, ub)


def _stage_c_body(e_ref, We, mb, Ue, ub, ee_ref, eu_ref):
    e = e_ref[...]
    ee_ref[...] = e @ We[...] + mb[...]
    eu_ref[...] = e @ Ue[...] + ub[...]


def _tc_stage_c(e, We, mb, Ue, ub):
    out_shapes = (
        jax.ShapeDtypeStruct((E, D), _f32),
        jax.ShapeDtypeStruct((E, EC), _f32),
    )
    return pl.pallas_call(
        _stage_c_body,
        grid=(_GE,),
        in_specs=[
            _rows((_REB, EC)),
            _full((EC, D)), _full((1, D)), _full((EC, EC)), _full((1, EC)),
        ],
        out_specs=[_rows((_REB, D)), _rows((_REB, EC))],
        out_shape=out_shapes,
    )(e, We, mb, Ue, ub)


def _final_body(h_ref, agg_ref, gb_ref, b_ref, Uh, Ua, dW1, db1, dW2, db2,
                out_ref):
    h = h_ref[...]
    agg = agg_ref[0] + agg_ref[1]
    oh = _onehot(b_ref[...])
    gbx = oh @ gb_ref[...]
    hn = h + jnp.maximum(h @ Uh[...] + agg @ Ua[...] + gbx, 0.0)
    o = jnp.maximum(hn @ dW1[...] + db1[...], 0.0)
    out_ref[...] = o @ dW2[...] + db2[...]


def _tc_final(h, agg2, gb, batch2, Uh, Ua, dW1, db1, dW2, db2):
    return pl.pallas_call(
        _final_body,
        grid=(_GN,),
        in_specs=[
            _rows((_RN, D)),
            pl.BlockSpec((2, _RN, D), lambda i: (0, i, 0)),
            _full((B, D)),
            _rows((_RN, 1)),
            _full((D, D)), _full((D, D)),
            _full((D, D)), _full((1, D)), _full((D, 4)), _full((1, 4)),
        ],
        out_specs=[_rows((_RN, 4))],
        out_shape=(jax.ShapeDtypeStruct((N, 4), _f32),),
    )(h, agg2, gb, batch2, Uh, Ua, dW1, db1, dW2, db2)[0]


@functools.cache
def _mesh():
    return plsc.VectorSubcoreMesh(core_axis_name="c", subcore_axis_name="s",
                                  num_cores=NCORES, num_subcores=NSUB)


def _sc_agg_body(hs_hbm, hd_hbm, ee_hbm, src_hbm, dst_hbm, z_hbm, out_hbm,
                 srcv, dstv, bs, bd, be, aggsh, sem1, sem2, sem3):
    c = lax.axis_index("c")
    s = lax.axis_index("s")
    row0 = s * ROWS_PER_TILE
    pltpu.sync_copy(z_hbm, aggsh.at[pl.ds(row0, ROWS_PER_TILE)])
    plsc.subcore_barrier()
    base = c * (E // NCORES) + s * PER_W

    def chunk(t, carry):
        off = base + t * CH
        pltpu.sync_copy(src_hbm.at[pl.ds(off, CH)], srcv)
        pltpu.sync_copy(dst_hbm.at[pl.ds(off, CH)], dstv)
        cp1 = pltpu.async_copy(hs_hbm.at[srcv], bs, sem1)
        cp2 = pltpu.async_copy(hd_hbm.at[dstv], bd, sem2)
        cp3 = pltpu.async_copy(ee_hbm.at[pl.ds(off, CH)], be, sem3)
        cp1.wait()
        cp2.wait()
        cp3.wait()

        def edge(i, carry2):
            for j in range(D // 16):
                sl = pl.ds(j * 16, 16)
                be[i, sl] = jnp.maximum(bs[i, sl] + bd[i, sl] + be[i, sl], 0.0)
            return carry2

        lax.fori_loop(0, CH, edge, 0, unroll=2)
        pltpu.sync_copy(be, aggsh.at[dstv], add=True)
        return carry

    lax.fori_loop(0, NCH, chunk, 0)
    plsc.subcore_barrier()
    pltpu.sync_copy(aggsh.at[pl.ds(row0, ROWS_PER_TILE)],
                    out_hbm.at[c, pl.ds(row0, ROWS_PER_TILE)])


@functools.cache
def _sc_agg_kernel():
    return pl.kernel(
        _sc_agg_body,
        out_type=jax.ShapeDtypeStruct((NCORES, NPAD, D), _f32),
        mesh=_mesh(),
        scratch_types=[
            pltpu.VMEM((CH,), jnp.int32),
            pltpu.VMEM((CH,), jnp.int32),
            pltpu.VMEM((CH, D), _f32),
            pltpu.VMEM((CH, D), _f32),
            pltpu.VMEM((CH, D), _f32),
            pltpu.VMEM_SHARED((NPAD, D), _f32),
            pltpu.SemaphoreType.DMA,
            pltpu.SemaphoreType.DMA,
            pltpu.SemaphoreType.DMA,
        ],
    )


def _sc_agg(hs, hd, ee, src, dst, zeros):
    return _sc_agg_kernel()(hs, hd, ee, src, dst, zeros)


def _sc_eupd_body(eu_hbm, us_hbm, ud_hbm, src_hbm, dst_hbm, out_hbm,
                  srcv, dstv, bs, bd, bu, sem1, sem2, sem3):
    c = lax.axis_index("c")
    s = lax.axis_index("s")
    base = c * (E // NCORES) + s * PER_W

    def chunk(t, carry):
        off = base + t * CH
        pltpu.sync_copy(src_hbm.at[pl.ds(off, CH)], srcv)
        pltpu.sync_copy(dst_hbm.at[pl.ds(off, CH)], dstv)
        cp1 = pltpu.async_copy(us_hbm.at[srcv], bs, sem1)
        cp2 = pltpu.async_copy(ud_hbm.at[dstv], bd, sem2)
        cp3 = pltpu.async_copy(eu_hbm.at[pl.ds(off, CH)], bu, sem3)
        cp1.wait()
        cp2.wait()
        cp3.wait()

        def edge(i, carry2):
            for j in range(EC // 16):
                sl = pl.ds(j * 16, 16)
                bu[i, sl] = jnp.maximum(bs[i, sl] + bd[i, sl] + bu[i, sl], 0.0)
            return carry2

        lax.fori_loop(0, CH, edge, 0, unroll=4)
        pltpu.sync_copy(bu, out_hbm.at[pl.ds(off, CH)])
        return carry

    lax.fori_loop(0, NCH, chunk, 0)


@functools.cache
def _sc_eupd_kernel():
    return pl.kernel(
        _sc_eupd_body,
        out_type=jax.ShapeDtypeStruct((E, EC), _f32),
        mesh=_mesh(),
        compiler_params=pltpu.CompilerParams(use_tc_tiling_on_sc=False),
        scratch_types=[
            pltpu.VMEM((CH,), jnp.int32),
            pltpu.VMEM((CH,), jnp.int32),
            pltpu.VMEM((CH, EC), _f32),
            pltpu.VMEM((CH, EC), _f32),
            pltpu.VMEM((CH, EC), _f32),
            pltpu.SemaphoreType.DMA,
            pltpu.SemaphoreType.DMA,
            pltpu.SemaphoreType.DMA,
        ],
    )


def _sc_eupd(eu, us, ud, src, dst):
    return _sc_eupd_kernel()(eu, us, ud, src, dst)


def kernel(x, x_mask, edge_index, edge_attr, pos, batch,
           enc_W1, enc_b1, enc_W2, enc_b2, eenc_W, eenc_b,
           msg_W, msg_b, upd_W, upd_b, eupd_W, eupd_b,
           dec_W1, dec_b1, dec_W2, dec_b2):
    src = edge_index[0]
    dst = edge_index[1]
    xin = jnp.concatenate([x, x_mask], axis=1)
    batch2 = batch[:, None]

    Ws, Wd, We = msg_W[:D], msg_W[D:2 * D], msg_W[2 * D:]
    Uh, Ua, Ug, Ub = (upd_W[:D], upd_W[D:2 * D], upd_W[2 * D:3 * D],
                      upd_W[3 * D:])
    Ue, Us, Ud = eupd_W[:EC], eupd_W[EC:EC + D], eupd_W[EC + D:]

    r1 = lambda v: v[None, :]
    zeros = jnp.zeros((ROWS_PER_TILE, D), _f32)

    h, hs, hd, gb, cbc, cnt = _tc_init_node(
        xin, x_mask, batch2, enc_W1, r1(enc_b1), enc_W2, r1(enc_b2),
        Ws, Wd, Ug, Ub, r1(upd_b))
    ee, eu = _tc_init_edge(edge_attr, eenc_W, r1(eenc_b), We, r1(msg_b),
                           Ue, r1(eupd_b))

    for r in range(REPEATS):
        agg2 = _sc_agg(hs, hd, ee, src, dst, zeros)
        if r < REPEATS - 1:
            h, hs, hd, us, ud, gb = _tc_stage_a(
                h, agg2, gb, cnt, cbc, batch2, Uh, Ua, Us, Ud, Ws, Wd, Ug)
            e = _sc_eupd(eu, us, ud, src, dst)
            ee, eu = _tc_stage_c(e, We, r1(msg_b), Ue, r1(eupd_b))
        else:
            out = _tc_final(h, agg2, gb, batch2, Uh, Ua,
                            dec_W1, r1(dec_b1), dec_W2, r1(dec_b2))
    return out


# speedup vs baseline: 1.9503x; 1.2712x over previous
import functools

import jax
import jax.numpy as jnp
from jax import lax
from jax.experimental import pallas as pl
from jax.experimental.pallas import tpu as pltpu
from jax.experimental.pallas import tpu_sc as plsc

N = 10000
E = 320000
B = 8
D = 128
EC = 32
REPEATS = 4

NCORES = 2
NSUB = 16
NW = NCORES * NSUB
CH = 80
PER_W = E // NW
NCH = PER_W // CH
NPAD = 10240
ROWS_PER_TILE = NPAD // NSUB
CHA = 40
NCHA = PER_W // CHA
KSB = 10
NSB = NCHA // KSB

_f32 = jnp.float32


_RN = 2000
_GN = N // _RN
_REB = 8000
_GE = E // _REB


def _onehot(b_block):
    return (b_block == lax.broadcasted_iota(jnp.int32, (1, B), 1)).astype(_f32)


def _full(shape):
    return pl.BlockSpec(shape, lambda i: (0,) * len(shape))


def _rows(shape):
    return pl.BlockSpec(shape, lambda i: (i,) + (0,) * (len(shape) - 1))


def _init_node_body(xin_ref, xm_ref, b_ref, W1, b1, W2, b2, Ws, Wd, Ug, Ub, ub,
                    h_ref, hs_ref, hd_ref, gb_ref, cbc_ref, cnt_ref,
                    xg_s, xbc_s, cnt_s, bcc_s):
    i = pl.program_id(0)
    t = jnp.maximum(xin_ref[...] @ W1[...] + b1[...], 0.0)
    h = t @ W2[...] + b2[...]
    h_ref[...] = h
    hs_ref[...] = h @ Ws[...]
    hd_ref[...] = h @ Wd[...]
    oh = _onehot(b_ref[...])
    bc = (xm_ref[...][:, B - 1:B] > 0.5).astype(_f32)
    ones = jnp.ones((_RN, D), _f32)
    ct = lax.dot_general(oh, ones, (((0,), (0,)), ((), ())))
    xg = lax.dot_general(oh, h, (((0,), (0,)), ((), ())))
    ohbc = oh * bc
    bct = lax.dot_general(ohbc, ones, (((0,), (0,)), ((), ())))
    xbc = lax.dot_general(ohbc, h, (((0,), (0,)), ((), ())))

    @pl.when(i == 0)
    def _():
        xg_s[...] = jnp.zeros_like(xg_s)
        xbc_s[...] = jnp.zeros_like(xbc_s)
        cnt_s[...] = jnp.zeros_like(cnt_s)
        bcc_s[...] = jnp.zeros_like(bcc_s)

    xg_s[...] += xg
    xbc_s[...] += xbc
    cnt_s[...] += ct
    bcc_s[...] += bct

    x_graph = xg_s[...] / jnp.maximum(cnt_s[...], 1.0)
    x_BC = xbc_s[...] / jnp.maximum(bcc_s[...], 1.0)
    cBC = x_BC @ Ub[...] + ub[...]
    gb_ref[...] = x_graph @ Ug[...] + cBC
    cbc_ref[...] = cBC
    cnt_ref[...] = cnt_s[...]


def _tc_init_node(xin, x_mask, batch2, W1, b1, W2, b2, Ws, Wd, Ug, Ub, ub):
    out_shapes = (
        jax.ShapeDtypeStruct((N, D), _f32),
        jax.ShapeDtypeStruct((N, D), _f32),
        jax.ShapeDtypeStruct((N, D), _f32),
        jax.ShapeDtypeStruct((B, D), _f32),
        jax.ShapeDtypeStruct((B, D), _f32),
        jax.ShapeDtypeStruct((B, D), _f32),
    )
    return pl.pallas_call(
        _init_node_body,
        grid=(_GN,),
        in_specs=[
            _rows((_RN, D)), _rows((_RN, B)), _rows((_RN, 1)),
            _full((D, D)), _full((1, D)), _full((D, D)), _full((1, D)),
            _full((D, D)), _full((D, D)), _full((D, D)), _full((D, D)),
            _full((1, D)),
        ],
        out_specs=[
            _rows((_RN, D)), _rows((_RN, D)), _rows((_RN, D)),
            _full((B, D)), _full((B, D)), _full((B, D)),
        ],
        out_shape=out_shapes,
        scratch_shapes=[pltpu.VMEM((B, D), _f32)] * 4,
    )(xin, x_mask, batch2, W1, b1, W2, b2, Ws, Wd, Ug, Ub, ub)


def _stage_a_body(h_ref, agg_ref, gb_ref, cnt_ref, cbc_ref, b_ref,
                  Uh, Ua, Us, Ud, Ws, Wd, Ug,
                  hn_ref, hs_ref, hd_ref, us_ref, ud_ref, gbn_ref, xg_s):
    i = pl.program_id(0)
    h = h_ref[...]
    agg = agg_ref[0] + agg_ref[1]
    oh = _onehot(b_ref[...])
    gbx = oh @ gb_ref[...]
    u = jnp.maximum(h @ Uh[...] + agg @ Ua[...] + gbx, 0.0)
    hn = h + u
    hn_ref[...] = hn
    hs_ref[...] = hn @ Ws[...]
    hd_ref[...] = hn @ Wd[...]
    us_ref[...] = hn @ Us[...]
    ud_ref[...] = hn @ Ud[...]
    xg = lax.dot_general(oh, hn, (((0,), (0,)), ((), ())))

    @pl.when(i == 0)
    def _():
        xg_s[...] = jnp.zeros_like(xg_s)

    xg_s[...] += xg
    x_graph = xg_s[...] / jnp.maximum(cnt_ref[...], 1.0)
    gbn_ref[...] = x_graph @ Ug[...] + cbc_ref[...]


def _tc_stage_a(h, agg2, gb, cnt, cbc, batch2, Uh, Ua, Us, Ud, Ws, Wd, Ug):
    out_shapes = (
        jax.ShapeDtypeStruct((N, D), _f32),
        jax.ShapeDtypeStruct((N, D), _f32),
        jax.ShapeDtypeStruct((N, D), _f32),
        jax.ShapeDtypeStruct((N, EC), _f32),
        jax.ShapeDtypeStruct((N, EC), _f32),
        jax.ShapeDtypeStruct((B, D), _f32),
    )
    return pl.pallas_call(
        _stage_a_body,
        grid=(_GN,),
        in_specs=[
            _rows((_RN, D)),
            pl.BlockSpec((2, _RN, D), lambda i: (0, i, 0)),
            _full((B, D)), _full((B, D)), _full((B, D)),
            _rows((_RN, 1)),
            _full((D, D)), _full((D, D)), _full((D, EC)), _full((D, EC)),
            _full((D, D)), _full((D, D)), _full((D, D)),
        ],
        out_specs=[
            _rows((_RN, D)), _rows((_RN, D)), _rows((_RN, D)),
            _rows((_RN, EC)), _rows((_RN, EC)), _full((B, D)),
        ],
        out_shape=out_shapes,
        scratch_shapes=[pltpu.VMEM((B, D), _f32)],
    )(h, agg2, gb, cnt, cbc, batch2, Uh, Ua, Us, Ud, Ws, Wd, Ug)


def _init_edge_body(ea_ref, eW, eb, We, mb, Ue, ub, ee_ref, eu_ref):
    e = jnp.maximum(ea_ref[...] @ eW[...] + eb[...], 0.0)
    ee_ref[...] = e @ We[...] + mb[...]
    eu_ref[...] = e @ Ue[...] + ub[...]


def _tc_init_edge(edge_attr, eW, eb, We, mb, Ue, ub):
    out_shapes = (
        jax.ShapeDtypeStruct((E, D), _f32),
        jax.ShapeDtypeStruct((E, EC), _f32),
    )
    return pl.pallas_call(
        _init_edge_body,
        grid=(_GE,),
        in_specs=[
            _rows((_REB, 4)),
            _full((4, EC)), _full((1, EC)), _full((EC, D)), _full((1, D)),
            _full((EC, EC)), _full((1, EC)),
        ],
        out_specs=[_rows((_REB, D)), _rows((_REB, EC))],
        out_shape=out_shapes,
    )(edge_attr, eW, eb, We, mb, Ue, ub)


def _stage_c_body(e_ref, We, mb, Ue, ub, ee_ref, eu_ref):
    e = e_ref[...]
    ee_ref[...] = e @ We[...] + mb[...]
    eu_ref[...] = e @ Ue[...] + ub[...]


def _tc_stage_c(e, We, mb, Ue, ub):
    out_shapes = (
        jax.ShapeDtypeStruct((E, D), _f32),
        jax.ShapeDtypeStruct((E, EC), _f32),
    )
    return pl.pallas_call(
        _stage_c_body,
        grid=(_GE,),
        in_specs=[
            _rows((_REB, EC)),
            _full((EC, D)), _full((1, D)), _full((EC, EC)), _full((1, EC)),
        ],
        out_specs=[_rows((_REB, D)), _rows((_REB, EC))],
        out_shape=out_shapes,
    )(e, We, mb, Ue, ub)


def _final_body(h_ref, agg_ref, gb_ref, b_ref, Uh, Ua, dW1, db1, dW2, db2,
                out_ref):
    h = h_ref[...]
    agg = agg_ref[0] + agg_ref[1]
    oh = _onehot(b_ref[...])
    gbx = oh @ gb_ref[...]
    hn = h + jnp.maximum(h @ Uh[...] + agg @ Ua[...] + gbx, 0.0)
    o = jnp.maximum(hn @ dW1[...] + db1[...], 0.0)
    out_ref[...] = o @ dW2[...] + db2[...]


def _tc_final(h, agg2, gb, batch2, Uh, Ua, dW1, db1, dW2, db2):
    return pl.pallas_call(
        _final_body,
        grid=(_GN,),
        in_specs=[
            _rows((_RN, D)),
            pl.BlockSpec((2, _RN, D), lambda i: (0, i, 0)),
            _full((B, D)),
            _rows((_RN, 1)),
            _full((D, D)), _full((D, D)),
            _full((D, D)), _full((1, D)), _full((D, 4)), _full((1, 4)),
        ],
        out_specs=[_rows((_RN, 4))],
        out_shape=(jax.ShapeDtypeStruct((N, 4), _f32),),
    )(h, agg2, gb, batch2, Uh, Ua, dW1, db1, dW2, db2)[0]


@functools.cache
def _mesh():
    return plsc.VectorSubcoreMesh(core_axis_name="c", subcore_axis_name="s",
                                  num_cores=NCORES, num_subcores=NSUB)


def _sc_agg_body(hs_hbm, hd_hbm, ee_hbm, src_hbm, dst_hbm, z_hbm, out_hbm,
                 srcv0, srcv1, dstv0, dstv1, bs0, bs1, bd0, bd1, be0, be1,
                 aggsh, semi0, semi1, semg0, semg1):
    c = lax.axis_index("c")
    s = lax.axis_index("s")
    row0 = s * ROWS_PER_TILE
    pltpu.sync_copy(z_hbm, aggsh.at[pl.ds(row0, ROWS_PER_TILE)])
    plsc.subcore_barrier()
    base = c * (E // NCORES) + s * PER_W

    srcv = (srcv0, srcv1)
    dstv = (dstv0, dstv1)
    bs = (bs0, bs1)
    bd = (bd0, bd1)
    be = (be0, be1)
    semi = (semi0, semi1)
    semg = (semg0, semg1)

    def _issue_gathers(p, off):
        pltpu.async_copy(hs_hbm.at[srcv[p]], bs[p], semg[p])
        pltpu.async_copy(hd_hbm.at[dstv[p]], bd[p], semg[p])
        pltpu.async_copy(ee_hbm.at[pl.ds(off, CHA)], be[p], semg[p])

    def _wait_gathers(p, off):
        pltpu.make_async_copy(hs_hbm.at[srcv[p]], bs[p], semg[p]).wait()
        pltpu.make_async_copy(hd_hbm.at[dstv[p]], bd[p], semg[p]).wait()
        pltpu.make_async_copy(ee_hbm.at[pl.ds(off, CHA)], be[p], semg[p]).wait()

    def _issue_idx(p, off):
        pltpu.async_copy(src_hbm.at[pl.ds(off, CHA)], srcv[p], semi[p])
        pltpu.async_copy(dst_hbm.at[pl.ds(off, CHA)], dstv[p], semi[p])

    def _wait_idx(p, off):
        pltpu.make_async_copy(src_hbm.at[pl.ds(off, CHA)], srcv[p], semi[p]).wait()
        pltpu.make_async_copy(dst_hbm.at[pl.ds(off, CHA)], dstv[p], semi[p]).wait()

    pltpu.sync_copy(src_hbm.at[pl.ds(base, CHA)], srcv[0])
    pltpu.sync_copy(dst_hbm.at[pl.ds(base, CHA)], dstv[0])
    _issue_gathers(0, base)
    _issue_idx(1, base + CHA)

    def _step(p, t):
        off = base + t * CHA
        _wait_gathers(p, off)

        @pl.when(t + 1 < NCHA)
        def _():
            _wait_idx(1 - p, off + CHA)
            _issue_gathers(1 - p, off + CHA)

        def edge(i, carry2):
            for j in range(D // 16):
                sl = pl.ds(j * 16, 16)
                be[p][i, sl] = jnp.maximum(
                    bs[p][i, sl] + bd[p][i, sl] + be[p][i, sl], 0.0)
            return carry2

        lax.fori_loop(0, CHA, edge, 0, unroll=4)
        pltpu.sync_copy(be[p], aggsh.at[dstv[p]], add=True)

        @pl.when(t + 2 < NCHA)
        def _():
            _issue_idx(p, off + 2 * CHA)

    def chunk(t, carry):
        @pl.when(t % 2 == 0)
        def _():
            _step(0, t)

        @pl.when(t % 2 == 1)
        def _():
            _step(1, t)

        return carry

    lax.fori_loop(0, NCHA, chunk, 0)
    plsc.subcore_barrier()
    pltpu.sync_copy(aggsh.at[pl.ds(row0, ROWS_PER_TILE)],
                    out_hbm.at[c, pl.ds(row0, ROWS_PER_TILE)])


@functools.cache
def _sc_agg_kernel():
    return pl.kernel(
        _sc_agg_body,
        out_type=jax.ShapeDtypeStruct((NCORES, NPAD, D), _f32),
        mesh=_mesh(),
        scratch_types=(
            [pltpu.VMEM((CHA,), jnp.int32)] * 4
            + [pltpu.VMEM((CHA, D), _f32)] * 6
            + [pltpu.VMEM_SHARED((NPAD, D), _f32)]
            + [pltpu.SemaphoreType.DMA] * 4
        ),
    )


def _sc_agg(hs, hd, ee, src, dst, zeros):
    return _sc_agg_kernel()(hs, hd, ee, src, dst, zeros)


def _sc_eupd_body(eu_hbm, us_hbm, ud_hbm, src_hbm, dst_hbm, out_hbm,
                  srcv0, srcv1, dstv0, dstv1, bs0, bs1, bd0, bd1, bu0, bu1,
                  semi0, semi1, semg0, semg1):
    c = lax.axis_index("c")
    s = lax.axis_index("s")
    base = c * (E // NCORES) + s * PER_W

    srcv = (srcv0, srcv1)
    dstv = (dstv0, dstv1)
    bs = (bs0, bs1)
    bd = (bd0, bd1)
    bu = (bu0, bu1)
    semi = (semi0, semi1)
    semg = (semg0, semg1)

    def _issue_gathers(p, off):
        pltpu.async_copy(us_hbm.at[srcv[p]], bs[p], semg[p])
        pltpu.async_copy(ud_hbm.at[dstv[p]], bd[p], semg[p])
        pltpu.async_copy(eu_hbm.at[pl.ds(off, CH)], bu[p], semg[p])

    def _wait_gathers(p, off):
        pltpu.make_async_copy(us_hbm.at[srcv[p]], bs[p], semg[p]).wait()
        pltpu.make_async_copy(ud_hbm.at[dstv[p]], bd[p], semg[p]).wait()
        pltpu.make_async_copy(eu_hbm.at[pl.ds(off, CH)], bu[p], semg[p]).wait()

    def _issue_idx(p, off):
        pltpu.async_copy(src_hbm.at[pl.ds(off, CH)], srcv[p], semi[p])
        pltpu.async_copy(dst_hbm.at[pl.ds(off, CH)], dstv[p], semi[p])

    def _wait_idx(p, off):
        pltpu.make_async_copy(src_hbm.at[pl.ds(off, CH)], srcv[p], semi[p]).wait()
        pltpu.make_async_copy(dst_hbm.at[pl.ds(off, CH)], dstv[p], semi[p]).wait()

    pltpu.sync_copy(src_hbm.at[pl.ds(base, CH)], srcv[0])
    pltpu.sync_copy(dst_hbm.at[pl.ds(base, CH)], dstv[0])
    _issue_gathers(0, base)
    _issue_idx(1, base + CH)

    def _step(p, t):
        off = base + t * CH
        _wait_gathers(p, off)

        @pl.when(t + 1 < NCH)
        def _():
            _wait_idx(1 - p, off + CH)
            _issue_gathers(1 - p, off + CH)

        def edge(i, carry2):
            for j in range(EC // 16):
                sl = pl.ds(j * 16, 16)
                bu[p][i, sl] = jnp.maximum(
                    bs[p][i, sl] + bd[p][i, sl] + bu[p][i, sl], 0.0)
            return carry2

        lax.fori_loop(0, CH, edge, 0, unroll=8)
        pltpu.sync_copy(bu[p], out_hbm.at[pl.ds(off, CH)])

        @pl.when(t + 2 < NCH)
        def _():
            _issue_idx(p, off + 2 * CH)

    def chunk(t, carry):
        @pl.when(t % 2 == 0)
        def _():
            _step(0, t)

        @pl.when(t % 2 == 1)
        def _():
            _step(1, t)

        return carry

    lax.fori_loop(0, NCH, chunk, 0)


@functools.cache
def _sc_eupd_kernel():
    return pl.kernel(
        _sc_eupd_body,
        out_type=jax.ShapeDtypeStruct((E, EC), _f32),
        mesh=_mesh(),
        compiler_params=pltpu.CompilerParams(use_tc_tiling_on_sc=False),
        scratch_types=(
            [pltpu.VMEM((CH,), jnp.int32)] * 4
            + [pltpu.VMEM((CH, EC), _f32)] * 6
            + [pltpu.SemaphoreType.DMA] * 4
        ),
    )


def _sc_eupd(eu, us, ud, src, dst):
    return _sc_eupd_kernel()(eu, us, ud, src, dst)


def kernel(x, x_mask, edge_index, edge_attr, pos, batch,
           enc_W1, enc_b1, enc_W2, enc_b2, eenc_W, eenc_b,
           msg_W, msg_b, upd_W, upd_b, eupd_W, eupd_b,
           dec_W1, dec_b1, dec_W2, dec_b2):
    src = edge_index[0]
    dst = edge_index[1]
    xin = jnp.concatenate([x, x_mask], axis=1)
    batch2 = batch[:, None]

    Ws, Wd, We = msg_W[:D], msg_W[D:2 * D], msg_W[2 * D:]
    Uh, Ua, Ug, Ub = (upd_W[:D], upd_W[D:2 * D], upd_W[2 * D:3 * D],
                      upd_W[3 * D:])
    Ue, Us, Ud = eupd_W[:EC], eupd_W[EC:EC + D], eupd_W[EC + D:]

    r1 = lambda v: v[None, :]
    zeros = jnp.zeros((ROWS_PER_TILE, D), _f32)

    h, hs, hd, gb, cbc, cnt = _tc_init_node(
        xin, x_mask, batch2, enc_W1, r1(enc_b1), enc_W2, r1(enc_b2),
        Ws, Wd, Ug, Ub, r1(upd_b))
    ee, eu = _tc_init_edge(edge_attr, eenc_W, r1(eenc_b), We, r1(msg_b),
                           Ue, r1(eupd_b))

    for r in range(REPEATS):
        agg2 = _sc_agg(hs, hd, ee, src, dst, zeros)
        if r < REPEATS - 1:
            h, hs, hd, us, ud, gb = _tc_stage_a(
                h, agg2, gb, cnt, cbc, batch2, Uh, Ua, Us, Ud, Ws, Wd, Ug)
            e = _sc_eupd(eu, us, ud, src, dst)
            ee, eu = _tc_stage_c(e, We, r1(msg_b), Ue, r1(eupd_b))
        else:
            out = _tc_final(h, agg2, gb, batch2, Uh, Ua,
                            dec_W1, r1(dec_b1), dec_W2, r1(dec_b2))
    return out
